# Initial kernel scaffold; baseline (speedup 1.0000x reference)
import jax
import jax.numpy as jnp
from jax.experimental import pallas as pl


def kernel(token_ids, weight):
    raise NotImplementedError("write your pallas kernel here")


# speedup vs baseline: 3.9562x; 3.9562x over previous
import functools

import jax
import jax.numpy as jnp
from jax import lax
from jax.experimental import pallas as pl
from jax.experimental.pallas import tpu as pltpu
from jax.experimental.pallas import tpu_sc as plsc

VOCAB = 100000
D_MODEL = 64
BATCH = 4096
HIST = 200
B_TOTAL = BATCH * HIST

_INFO = plsc.get_sparse_core_info()
_NC = _INFO.num_cores
_NS = _INFO.num_subcores
_NW = _NC * _NS
_B_PER_W = B_TOTAL // _NW
_CHUNK = 512
_NCHUNK = _B_PER_W // _CHUNK


def _emb_body(tok_hbm, w_hbm, out_hbm, idx_v, rows_v, gsem):
  wid = lax.axis_index("s") * _NC + lax.axis_index("c")
  base = wid * _B_PER_W

  @pl.loop(0, _NCHUNK)
  def _chunk(i):
    off = base + i * _CHUNK
    pltpu.sync_copy(tok_hbm.at[pl.ds(off, _CHUNK)], idx_v)
    pltpu.async_copy(w_hbm.at[idx_v], rows_v, gsem).wait()
    pltpu.sync_copy(rows_v, out_hbm.at[pl.ds(off, _CHUNK)])


_emb = functools.partial(
    pl.kernel,
    out_type=jax.ShapeDtypeStruct((B_TOTAL, D_MODEL), jnp.float32),
    mesh=plsc.VectorSubcoreMesh(core_axis_name="c", subcore_axis_name="s"),
    scratch_types=[
        pltpu.VMEM((_CHUNK,), jnp.int32),
        pltpu.VMEM((_CHUNK, D_MODEL), jnp.float32),
        pltpu.SemaphoreType.DMA,
    ],
    compiler_params=pltpu.CompilerParams(use_tc_tiling_on_sc=False),
)(_emb_body)


@jax.jit
def kernel(token_ids, weight):
  tok = token_ids.reshape(B_TOTAL).astype(jnp.int32)
  out = _emb(tok, weight)
  return out.reshape(BATCH, HIST, D_MODEL)


# speedup vs baseline: 4.2618x; 1.0773x over previous
import functools

import jax
import jax.numpy as jnp
from jax import lax
from jax.experimental import pallas as pl
from jax.experimental.pallas import tpu as pltpu
from jax.experimental.pallas import tpu_sc as plsc

VOCAB = 100000
D_MODEL = 64
BATCH = 4096
HIST = 200
B_TOTAL = BATCH * HIST

_INFO = plsc.get_sparse_core_info()
_NC = _INFO.num_cores
_NS = _INFO.num_subcores
_NW = _NC * _NS
_B_PER_W = B_TOTAL // _NW
_CHUNK = 256
_NCHUNK = _B_PER_W // _CHUNK
_NBUF = 4
_LOOK = 2


def _emb_body(tok_hbm, w_hbm, out_hbm, idx_all, rows_v, gsem, osem):
  wid = lax.axis_index("s") * _NC + lax.axis_index("c")
  base = wid * _B_PER_W
  pltpu.sync_copy(tok_hbm.at[pl.ds(base, _B_PER_W)], idx_all)

  def gather(g, b):
    return pltpu.make_async_copy(
        w_hbm.at[idx_all.at[pl.ds(g * _CHUNK, _CHUNK)]],
        rows_v.at[b], gsem.at[b])

  def store(g, b):
    return pltpu.make_async_copy(
        rows_v.at[b], out_hbm.at[pl.ds(base + g * _CHUNK, _CHUNK)],
        osem.at[b])

  for gp in range(_LOOK):
    gather(gp, gp).start()

  @pl.loop(0, _NCHUNK)
  def _chunk(g):
    b = lax.rem(g, _NBUF)
    gather(g, b).wait()
    store(g, b).start()
    gn = g + _LOOK

    @pl.when(gn < _NCHUNK)
    def _prefetch():
      bn = lax.rem(gn, _NBUF)

      @pl.when(g >= _LOOK)
      def _drain():
        store(g - _LOOK, bn).wait()

      gather(gn, bn).start()

  for j in range(2 * _LOOK):
    g = _NCHUNK - 2 * _LOOK + j
    store(g, g % _NBUF).wait()


_emb = functools.partial(
    pl.kernel,
    out_type=jax.ShapeDtypeStruct((B_TOTAL, D_MODEL), jnp.float32),
    mesh=plsc.VectorSubcoreMesh(core_axis_name="c", subcore_axis_name="s"),
    scratch_types=[
        pltpu.VMEM((_B_PER_W,), jnp.int32),
        pltpu.VMEM((_NBUF, _CHUNK, D_MODEL), jnp.float32),
        pltpu.SemaphoreType.DMA((_NBUF,)),
        pltpu.SemaphoreType.DMA((_NBUF,)),
    ],
    compiler_params=pltpu.CompilerParams(use_tc_tiling_on_sc=False),
)(_emb_body)


@jax.jit
def kernel(token_ids, weight):
  tok = token_ids.reshape(B_TOTAL).astype(jnp.int32)
  out = _emb(tok, weight)
  return out.reshape(BATCH, HIST, D_MODEL)
